# Initial kernel scaffold; baseline (speedup 1.0000x reference)
import jax
import jax.numpy as jnp
from jax.experimental import pallas as pl


def kernel(x, edge_index, shrink, att0, att1):
    raise NotImplementedError("write your pallas kernel here")


# speedup vs baseline: 2.4714x; 2.4714x over previous
import functools

import jax
import jax.numpy as jnp
from jax import lax
from jax.experimental import pallas as pl
from jax.experimental.pallas import tpu as pltpu

N = 10000
NPAD = 10240
E = 160000
EPAD = 163840
IN_DIM = 256
OUT_DIM = 256
N_HEADS = 4
NB = 1024
NPB = NPAD // NB


def _v_body(att0_ref, att1_ref, shrink_ref, v_ref):
    for h in range(N_HEADS):
        v_ref[h : h + 1, :] = att0_ref[h : h + 1, :] @ shrink_ref[h]
        v_ref[N_HEADS + h : N_HEADS + h + 1, :] = att1_ref[h : h + 1, :] @ shrink_ref[h]


def _proj_body(x_ref, shrink_ref, v_ref, a_ref, *table_refs):
    xb = x_ref[...]
    a_ref[...] = v_ref[...] @ xb
    for fb in range(8):
        h, half = fb // 2, fb % 2
        w = shrink_ref[h, half * 128 : (half + 1) * 128, :]
        table_refs[fb][...] = lax.dot_general(
            xb, w, (((0,), (1,)), ((), ()))
        )


def _final_body(acc_ref, o_ref):
    z = jnp.transpose(acc_ref[0], (1, 0))
    o_ref[...] = jnp.where(z > 0, z, jnp.exp(z) - 1.0)


def _tc_project(x_pad, shrink, att0, att1):
    v = pl.pallas_call(
        _v_body,
        out_shape=jax.ShapeDtypeStruct((8, IN_DIM), jnp.float32),
    )(att0, att1, shrink)

    out_shapes = (
        jax.ShapeDtypeStruct((8, NPAD), jnp.float32),
    ) + tuple(jax.ShapeDtypeStruct((NPAD, 128), jnp.float32) for _ in range(8))
    grid = (NPB,)
    in_specs = [
        pl.BlockSpec((IN_DIM, NB), lambda n: (0, n)),
        pl.BlockSpec((N_HEADS, IN_DIM, IN_DIM), lambda n: (0, 0, 0)),
        pl.BlockSpec((8, IN_DIM), lambda n: (0, 0)),
    ]
    out_specs = [pl.BlockSpec((8, NB), lambda n: (0, n))] + [
        pl.BlockSpec((NB, 128), lambda n: (n, 0)) for _ in range(8)
    ]
    a_all, *tables = pl.pallas_call(
        _proj_body,
        grid=grid,
        in_specs=in_specs,
        out_specs=out_specs,
        out_shape=out_shapes,
    )(x_pad, shrink, v)
    return a_all, tables


def _tc_finalize(acc):
    return pl.pallas_call(
        _final_body,
        grid=(8, NPB),
        in_specs=[pl.BlockSpec((1, NB, 128), lambda f, n: (f, n, 0))],
        out_specs=pl.BlockSpec((128, NB), lambda f, n: (f, n)),
        out_shape=jax.ShapeDtypeStruct((8 * 128, NPAD), jnp.float32),
    )(acc)


def _edge_phase_jax(rows, cols, a_all, tables):
    acc = []
    for fb in range(8):
        h = fb // 2
        e = a_all[h, cols] + a_all[N_HEADS + h, rows]
        ex = jnp.exp(e)
        denom = jax.ops.segment_sum(ex, rows, num_segments=NPAD)
        w = ex / denom[rows]
        msg = w[:, None] * tables[fb][cols]
        acc.append(jax.ops.segment_sum(msg, rows, num_segments=NPAD))
    return jnp.stack(acc, axis=0)


def kernel(x, edge_index, shrink, att0, att1):
    x = x.astype(jnp.float32)
    x_pad = jnp.pad(x, ((0, 0), (0, NPAD - N)))
    rows = edge_index[0].astype(jnp.int32)
    cols = edge_index[1].astype(jnp.int32)

    a_all, tables = _tc_project(x_pad, shrink, att0, att1)
    acc = _edge_phase_jax(rows, cols, a_all, tables)
    out = _tc_finalize(acc)
    return out[:, :N]


# speedup vs baseline: 6.2083x; 2.5120x over previous
import functools

import jax
import jax.numpy as jnp
from jax import lax
from jax.experimental import pallas as pl
from jax.experimental.pallas import tpu as pltpu
from jax.experimental.pallas import tpu_sc as plsc

N = 10000
NPAD = 10240
E = 160000
EPAD = 163840
IN_DIM = 256
OUT_DIM = 256
N_HEADS = 4
NB = 1024
NPB = NPAD // NB


def _v_body(att0_ref, att1_ref, shrink_ref, v_ref):
    for h in range(N_HEADS):
        v_ref[h : h + 1, :] = att0_ref[h : h + 1, :] @ shrink_ref[h]
        v_ref[N_HEADS + h : N_HEADS + h + 1, :] = att1_ref[h : h + 1, :] @ shrink_ref[h]


def _proj_body(x_ref, shrink_ref, v_ref, a_ref, *table_refs):
    xb = x_ref[...]
    a_ref[...] = v_ref[...] @ xb
    for fb in range(8):
        h, half = fb // 2, fb % 2
        w = shrink_ref[h, half * 128 : (half + 1) * 128, :]
        table_refs[fb][...] = lax.dot_general(
            xb, w, (((0,), (1,)), ((), ()))
        )


def _final_body(acc_ref, o_ref):
    a4 = acc_ref[...]
    z = jnp.concatenate([a4[0], a4[1], a4[2], a4[3]], axis=1)
    z = jnp.transpose(z, (1, 0))
    o_ref[...] = jnp.where(z > 0, z, jnp.exp(z) - 1.0)


def _tc_project(x_pad, shrink, att0, att1):
    v = pl.pallas_call(
        _v_body,
        out_shape=jax.ShapeDtypeStruct((8, IN_DIM), jnp.float32),
    )(att0, att1, shrink)

    out_shapes = (
        jax.ShapeDtypeStruct((8, NPAD), jnp.float32),
    ) + tuple(jax.ShapeDtypeStruct((NPAD, 128), jnp.float32) for _ in range(8))
    grid = (NPB,)
    in_specs = [
        pl.BlockSpec((IN_DIM, NB), lambda n: (0, n)),
        pl.BlockSpec((N_HEADS, IN_DIM, IN_DIM), lambda n: (0, 0, 0)),
        pl.BlockSpec((8, IN_DIM), lambda n: (0, 0)),
    ]
    out_specs = [pl.BlockSpec((8, NB), lambda n: (0, n))] + [
        pl.BlockSpec((NB, 128), lambda n: (n, 0)) for _ in range(8)
    ]
    a_all, *tables = pl.pallas_call(
        _proj_body,
        grid=grid,
        in_specs=in_specs,
        out_specs=out_specs,
        out_shape=out_shapes,
    )(x_pad, shrink, v)
    return a_all, tables


def _tc_finalize(acc):
    return pl.pallas_call(
        _final_body,
        grid=(8, NPB),
        in_specs=[pl.BlockSpec((4, NB, 32), lambda f, n: (f, n, 0))],
        out_specs=pl.BlockSpec((128, NB), lambda f, n: (f, n)),
        out_shape=jax.ShapeDtypeStruct((8 * 128, NPAD), jnp.float32),
    )(acc)


_EC = 80

_sc_mesh = plsc.VectorSubcoreMesh(core_axis_name="c", subcore_axis_name="s")


def _sc_edge_body(
    rows_hbm, cols_hbm, a_hbm,
    t0, t1, t2, t3, t4, t5, t6, t7,
    acc_out,
    ridx, cidx, cidx4, exb, a0loc, a1loc, gbuf, zacc,
    den_blk,
    acc_sh, sem,
):
    tables = (t0, t1, t2, t3, t4, t5, t6, t7)
    c = lax.axis_index("c")
    s = lax.axis_index("s")

    pltpu.sync_copy(rows_hbm.at[s], ridx)
    pltpu.sync_copy(cols_hbm.at[s], cidx)

    zero16f = jnp.zeros((16,), jnp.float32)
    for i in range(16):
        for j in range(2):
            zacc[i, pl.ds(j * 16, 16)] = zero16f

    for hl in range(2):
        h = 2 * c + hl
        pltpu.sync_copy(a_hbm.at[h], a0loc)
        pltpu.sync_copy(a_hbm.at[N_HEADS + h], a1loc)
        hsplat = jnp.full((16,), hl, jnp.int32)

        def zden_body(t, carry):
            pltpu.sync_copy(zacc, acc_sh.at[pl.ds(s * 640 + t * 16, 16)])
            return carry

        lax.fori_loop(0, 40, zden_body, 0)
        plsc.subcore_barrier()

        def p1_chunk(k, carry, hl=hl, hsplat=hsplat):
            for j in range(8):
                sl = pl.ds(j * 16, 16)
                vc = cidx[k, sl]
                vr = ridx[k, sl]
                va0 = plsc.load_gather(a0loc, [vc])
                va1 = plsc.load_gather(a1loc, [vr])
                exb[hl, pl.ds(k * 128 + j * 16, 16)] = jnp.exp(va0 + va1)
            def splat_body(e, c2, hl=hl, hsplat=hsplat, k=k):
                w = plsc.load_gather(
                    exb, [hsplat, jnp.full((16,), k * 128 + e, jnp.int32)]
                )
                gbuf[e, pl.ds(0, 16)] = w
                gbuf[e, pl.ds(16, 16)] = w
                return c2

            lax.fori_loop(0, 128, splat_body, 0)
            pltpu.sync_copy(gbuf, acc_sh.at[ridx.at[k]], add=True)
            return carry

        lax.fori_loop(0, _EC, p1_chunk, 0)
        plsc.subcore_barrier()
        for w in range(5):
            pltpu.sync_copy(acc_sh.at[pl.ds(s * 640 + w * 128, 128)], gbuf)

            def dsave_body(r, carry, w=w, hl=hl):
                den_blk[hl, w * 128 + r, :] = gbuf[r, pl.ds(0, 16)]
                return carry

            lax.fori_loop(0, 128, dsave_body, 0)
        plsc.subcore_barrier()

    for fb in range(8):
        cc = fb // 4
        hl = (fb % 4) // 2
        tref = tables[fb]

        @pl.when(c == cc)
        def _(fb=fb, hl=hl, tref=tref):
            def h2_body(h2, outer_carry, fb=fb, hl=hl, tref=tref):

                def prep_body(k, carry, h2=h2):
                    for j in range(8):
                        sl = pl.ds(j * 16, 16)
                        cidx4[k, sl] = cidx[k, sl] * 4 + h2
                    return carry

                lax.fori_loop(0, _EC, prep_body, 0)

                def z_body(t, carry):
                    pltpu.sync_copy(
                        zacc, acc_sh.at[pl.ds(s * 640 + t * 16, 16)]
                    )
                    return carry

                lax.fori_loop(0, 40, z_body, 0)
                plsc.subcore_barrier()
                hsplat = jnp.full((16,), hl, jnp.int32)

                def ch_body(k, carry, hl=hl, tref=tref, hsplat=hsplat):
                    pltpu.async_copy(tref.at[cidx4.at[k]], gbuf, sem).wait()

                    def e_body(e, c2):
                        w = plsc.load_gather(
                            exb,
                            [hsplat, jnp.full((16,), k * 128 + e, jnp.int32)],
                        )
                        for j in range(2):
                            sl = pl.ds(j * 16, 16)
                            gbuf[e, sl] = gbuf[e, sl] * w
                        return c2

                    lax.fori_loop(0, 128, e_body, 0)
                    pltpu.sync_copy(gbuf, acc_sh.at[ridx.at[k]], add=True)
                    return carry

                lax.fori_loop(0, _EC, ch_body, 0)
                plsc.subcore_barrier()

                for w in range(5):
                    pltpu.sync_copy(
                        acc_sh.at[pl.ds(s * 640 + w * 128, 128)], gbuf
                    )

                    def nrm_body(r, carry, w=w, hl=hl):
                        den16 = den_blk[hl, w * 128 + r, :]
                        inv = jnp.where(den16 > 0, 1.0 / den16, 0.0)
                        for j in range(2):
                            sl = pl.ds(j * 16, 16)
                            gbuf[r, sl] = gbuf[r, sl] * inv
                        return carry

                    lax.fori_loop(0, 128, nrm_body, 0)
                    pltpu.sync_copy(
                        gbuf,
                        acc_out.at[
                            4 * fb + h2, pl.ds(s * 640 + w * 128, 128)
                        ],
                    )
                plsc.subcore_barrier()
                return outer_carry

            lax.fori_loop(0, 4, h2_body, 0)


_sc_edge = functools.partial(
    pl.kernel,
    out_type=jax.ShapeDtypeStruct((32, NPAD, 32), jnp.float32),
    mesh=_sc_mesh,
    compiler_params=pltpu.CompilerParams(
        needs_layout_passes=False, use_tc_tiling_on_sc=False
    ),
    scratch_types=[
        pltpu.VMEM((_EC, 128), jnp.int32),
        pltpu.VMEM((_EC, 128), jnp.int32),
        pltpu.VMEM((_EC, 128), jnp.int32),
        pltpu.VMEM((2, NPAD), jnp.float32),
        pltpu.VMEM((NPAD,), jnp.float32),
        pltpu.VMEM((NPAD,), jnp.float32),
        pltpu.VMEM((128, 32), jnp.float32),
        pltpu.VMEM((16, 32), jnp.float32),
        pltpu.VMEM((2, 640, 16), jnp.float32),
        pltpu.VMEM_SHARED((NPAD, 32), jnp.float32),
        pltpu.SemaphoreType.DMA,
    ],
)(_sc_edge_body)


def kernel(x, edge_index, shrink, att0, att1):
    x = x.astype(jnp.float32)
    x_pad = jnp.pad(x, ((0, 0), (0, NPAD - N)))
    rows = edge_index[0].astype(jnp.int32)
    cols = edge_index[1].astype(jnp.int32)
    pad_idx = N + (jnp.arange(EPAD - E, dtype=jnp.int32) % (NPAD - N))
    rows_p = jnp.concatenate([rows, pad_idx]).reshape(16, _EC, 128)
    cols_p = jnp.concatenate([cols, pad_idx]).reshape(16, _EC, 128)

    a_all, tables = _tc_project(x_pad, shrink, att0, att1)
    tables2 = [t.reshape(4 * NPAD, 32) for t in tables]
    acc = _sc_edge(rows_p, cols_p, a_all, *tables2)
    out = _tc_finalize(acc)
    return out[:, :N]


# speedup vs baseline: 6.2279x; 1.0032x over previous
import functools

import jax
import jax.numpy as jnp
from jax import lax
from jax.experimental import pallas as pl
from jax.experimental.pallas import tpu as pltpu
from jax.experimental.pallas import tpu_sc as plsc

N = 10000
NPAD = 10240
E = 160000
EPAD = 163840
IN_DIM = 256
OUT_DIM = 256
N_HEADS = 4
NB = 1024
NPB = NPAD // NB


def _v_body(att0_ref, att1_ref, shrink_ref, v_ref):
    for h in range(N_HEADS):
        v_ref[h : h + 1, :] = att0_ref[h : h + 1, :] @ shrink_ref[h]
        v_ref[N_HEADS + h : N_HEADS + h + 1, :] = att1_ref[h : h + 1, :] @ shrink_ref[h]


def _proj_body(x_ref, shrink_ref, v_ref, a_ref, *table_refs):
    xb = x_ref[...]
    a_ref[...] = v_ref[...] @ xb
    for fb in range(8):
        h, half = fb // 2, fb % 2
        w = shrink_ref[h, half * 128 : (half + 1) * 128, :]
        table_refs[fb][...] = lax.dot_general(
            xb, w, (((0,), (1,)), ((), ()))
        )


def _final_body(acc_ref, o_ref):
    a4 = acc_ref[...]
    z = jnp.concatenate([a4[0], a4[1], a4[2], a4[3]], axis=1)
    z = jnp.transpose(z, (1, 0))
    o_ref[...] = jnp.where(z > 0, z, jnp.exp(z) - 1.0)


def _tc_project(x_pad, shrink, att0, att1):
    v = pl.pallas_call(
        _v_body,
        out_shape=jax.ShapeDtypeStruct((8, IN_DIM), jnp.float32),
    )(att0, att1, shrink)

    out_shapes = (
        jax.ShapeDtypeStruct((8, NPAD), jnp.float32),
    ) + tuple(jax.ShapeDtypeStruct((NPAD, 128), jnp.float32) for _ in range(8))
    grid = (NPB,)
    in_specs = [
        pl.BlockSpec((IN_DIM, NB), lambda n: (0, n)),
        pl.BlockSpec((N_HEADS, IN_DIM, IN_DIM), lambda n: (0, 0, 0)),
        pl.BlockSpec((8, IN_DIM), lambda n: (0, 0)),
    ]
    out_specs = [pl.BlockSpec((8, NB), lambda n: (0, n))] + [
        pl.BlockSpec((NB, 128), lambda n: (n, 0)) for _ in range(8)
    ]
    a_all, *tables = pl.pallas_call(
        _proj_body,
        grid=grid,
        in_specs=in_specs,
        out_specs=out_specs,
        out_shape=out_shapes,
    )(x_pad, shrink, v)
    return a_all, tables


def _tc_finalize(acc):
    return pl.pallas_call(
        _final_body,
        grid=(8, NPB),
        in_specs=[pl.BlockSpec((4, NB, 32), lambda f, n: (f, n, 0))],
        out_specs=pl.BlockSpec((128, NB), lambda f, n: (f, n)),
        out_shape=jax.ShapeDtypeStruct((8 * 128, NPAD), jnp.float32),
    )(acc)


_EC = 80

_sc_mesh = plsc.VectorSubcoreMesh(core_axis_name="c", subcore_axis_name="s")


def _sc_edge_body(
    rows_hbm, cols_hbm, a_hbm,
    t0, t1, t2, t3, t4, t5, t6, t7,
    acc_out,
    ridx, cidx, cidx4, exb, a0loc, a1loc, gbuf, zacc,
    den_blk,
    acc_sh, sem,
):
    tables = (t0, t1, t2, t3, t4, t5, t6, t7)
    c = lax.axis_index("c")
    s = lax.axis_index("s")

    pltpu.sync_copy(rows_hbm.at[s], ridx)
    pltpu.sync_copy(cols_hbm.at[s], cidx)

    zero16f = jnp.zeros((16,), jnp.float32)
    for i in range(16):
        for j in range(2):
            zacc[i, pl.ds(j * 16, 16)] = zero16f

    for hl in range(2):
        h = 2 * c + hl
        pltpu.sync_copy(a_hbm.at[h], a0loc)
        pltpu.sync_copy(a_hbm.at[N_HEADS + h], a1loc)
        hsplat = jnp.full((16,), hl, jnp.int32)

        def zden_body(t, carry):
            pltpu.sync_copy(zacc, acc_sh.at[pl.ds(s * 640 + t * 16, 16)])
            return carry

        lax.fori_loop(0, 40, zden_body, 0)
        plsc.subcore_barrier()

        def p1_chunk(k, carry, hl=hl, hsplat=hsplat):
            for j in range(8):
                sl = pl.ds(j * 16, 16)
                vc = cidx[k, sl]
                vr = ridx[k, sl]
                va0 = plsc.load_gather(a0loc, [vc])
                va1 = plsc.load_gather(a1loc, [vr])
                exb[hl, pl.ds(k * 128 + j * 16, 16)] = jnp.exp(va0 + va1)
            def splat_body(e, c2, hl=hl, hsplat=hsplat, k=k):
                for u in range(4):
                    w = plsc.load_gather(
                        exb,
                        [hsplat, jnp.full((16,), k * 128 + e * 4 + u, jnp.int32)],
                    )
                    gbuf[e * 4 + u, pl.ds(0, 16)] = w
                    gbuf[e * 4 + u, pl.ds(16, 16)] = w
                return c2

            lax.fori_loop(0, 32, splat_body, 0)
            pltpu.sync_copy(gbuf, acc_sh.at[ridx.at[k]], add=True)
            return carry

        lax.fori_loop(0, _EC, p1_chunk, 0)
        plsc.subcore_barrier()
        for w in range(5):
            pltpu.sync_copy(acc_sh.at[pl.ds(s * 640 + w * 128, 128)], gbuf)

            def dsave_body(r, carry, w=w, hl=hl):
                for u in range(4):
                    den_blk[hl, w * 128 + r * 4 + u, :] = gbuf[
                        r * 4 + u, pl.ds(0, 16)
                    ]
                return carry

            lax.fori_loop(0, 32, dsave_body, 0)
        plsc.subcore_barrier()

    for fb in range(8):
        cc = fb // 4
        hl = (fb % 4) // 2
        tref = tables[fb]

        @pl.when(c == cc)
        def _(fb=fb, hl=hl, tref=tref):
            def h2_body(h2, outer_carry, fb=fb, hl=hl, tref=tref):

                def prep_body(k, carry, h2=h2):
                    for j in range(8):
                        sl = pl.ds(j * 16, 16)
                        cidx4[k, sl] = cidx[k, sl] * 4 + h2
                    return carry

                lax.fori_loop(0, _EC, prep_body, 0)

                def z_body(t, carry):
                    pltpu.sync_copy(
                        zacc, acc_sh.at[pl.ds(s * 640 + t * 16, 16)]
                    )
                    return carry

                lax.fori_loop(0, 40, z_body, 0)
                plsc.subcore_barrier()
                hsplat = jnp.full((16,), hl, jnp.int32)

                def ch_body(k, carry, hl=hl, tref=tref, hsplat=hsplat):
                    pltpu.async_copy(tref.at[cidx4.at[k]], gbuf, sem).wait()

                    def e_body(e, c2):
                        for u in range(4):
                            w = plsc.load_gather(
                                exb,
                                [
                                    hsplat,
                                    jnp.full(
                                        (16,), k * 128 + e * 4 + u, jnp.int32
                                    ),
                                ],
                            )
                            for j in range(2):
                                sl = pl.ds(j * 16, 16)
                                gbuf[e * 4 + u, sl] = gbuf[e * 4 + u, sl] * w
                        return c2

                    lax.fori_loop(0, 32, e_body, 0)
                    pltpu.sync_copy(gbuf, acc_sh.at[ridx.at[k]], add=True)
                    return carry

                lax.fori_loop(0, _EC, ch_body, 0)
                plsc.subcore_barrier()

                for w in range(5):
                    pltpu.sync_copy(
                        acc_sh.at[pl.ds(s * 640 + w * 128, 128)], gbuf
                    )

                    def nrm_body(r, carry, w=w, hl=hl):
                        for u in range(4):
                            den16 = den_blk[hl, w * 128 + r * 4 + u, :]
                            inv = jnp.where(den16 > 0, 1.0 / den16, 0.0)
                            for j in range(2):
                                sl = pl.ds(j * 16, 16)
                                gbuf[r * 4 + u, sl] = gbuf[r * 4 + u, sl] * inv
                        return carry

                    lax.fori_loop(0, 32, nrm_body, 0)
                    pltpu.sync_copy(
                        gbuf,
                        acc_out.at[
                            4 * fb + h2, pl.ds(s * 640 + w * 128, 128)
                        ],
                    )
                plsc.subcore_barrier()
                return outer_carry

            lax.fori_loop(0, 4, h2_body, 0)


_sc_edge = functools.partial(
    pl.kernel,
    out_type=jax.ShapeDtypeStruct((32, NPAD, 32), jnp.float32),
    mesh=_sc_mesh,
    compiler_params=pltpu.CompilerParams(
        needs_layout_passes=False, use_tc_tiling_on_sc=False
    ),
    scratch_types=[
        pltpu.VMEM((_EC, 128), jnp.int32),
        pltpu.VMEM((_EC, 128), jnp.int32),
        pltpu.VMEM((_EC, 128), jnp.int32),
        pltpu.VMEM((2, NPAD), jnp.float32),
        pltpu.VMEM((NPAD,), jnp.float32),
        pltpu.VMEM((NPAD,), jnp.float32),
        pltpu.VMEM((128, 32), jnp.float32),
        pltpu.VMEM((16, 32), jnp.float32),
        pltpu.VMEM((2, 640, 16), jnp.float32),
        pltpu.VMEM_SHARED((NPAD, 32), jnp.float32),
        pltpu.SemaphoreType.DMA,
    ],
)(_sc_edge_body)


def kernel(x, edge_index, shrink, att0, att1):
    x = x.astype(jnp.float32)
    x_pad = jnp.pad(x, ((0, 0), (0, NPAD - N)))
    rows = edge_index[0].astype(jnp.int32)
    cols = edge_index[1].astype(jnp.int32)
    pad_idx = N + (jnp.arange(EPAD - E, dtype=jnp.int32) % (NPAD - N))
    rows_p = jnp.concatenate([rows, pad_idx]).reshape(16, _EC, 128)
    cols_p = jnp.concatenate([cols, pad_idx]).reshape(16, _EC, 128)

    a_all, tables = _tc_project(x_pad, shrink, att0, att1)
    tables2 = [t.reshape(4 * NPAD, 32) for t in tables]
    acc = _sc_edge(rows_p, cols_p, a_all, *tables2)
    out = _tc_finalize(acc)
    return out[:, :N]


# speedup vs baseline: 8.3474x; 1.3403x over previous
import functools

import jax
import jax.numpy as jnp
from jax import lax
from jax.experimental import pallas as pl
from jax.experimental.pallas import tpu as pltpu
from jax.experimental.pallas import tpu_sc as plsc

N = 10000
NPAD = 10240
E = 160000
EPAD = 163840
IN_DIM = 256
OUT_DIM = 256
N_HEADS = 4
NB = 1024
NPB = NPAD // NB


def _v_body(att0_ref, att1_ref, shrink_ref, v_ref):
    for h in range(N_HEADS):
        v_ref[h : h + 1, :] = att0_ref[h : h + 1, :] @ shrink_ref[h]
        v_ref[N_HEADS + h : N_HEADS + h + 1, :] = att1_ref[h : h + 1, :] @ shrink_ref[h]


def _proj_body(x_ref, shrink_ref, v_ref, a_ref, *table_refs):
    xb = x_ref[...]
    a_ref[...] = v_ref[...] @ xb
    for fb in range(8):
        h, half = fb // 2, fb % 2
        w = shrink_ref[h, half * 128 : (half + 1) * 128, :]
        table_refs[fb][...] = lax.dot_general(
            xb, w, (((0,), (1,)), ((), ()))
        )


def _final_body(acc_ref, o_ref):
    a4 = acc_ref[...]
    z = jnp.concatenate([a4[0], a4[1], a4[2], a4[3]], axis=1)
    z = jnp.transpose(z, (1, 0))
    o_ref[...] = jnp.where(z > 0, z, jnp.exp(z) - 1.0)


def _tc_project(x_pad, shrink, att0, att1):
    v = pl.pallas_call(
        _v_body,
        out_shape=jax.ShapeDtypeStruct((8, IN_DIM), jnp.float32),
    )(att0, att1, shrink)

    out_shapes = (
        jax.ShapeDtypeStruct((8, NPAD), jnp.float32),
    ) + tuple(jax.ShapeDtypeStruct((NPAD, 128), jnp.float32) for _ in range(8))
    grid = (NPB,)
    in_specs = [
        pl.BlockSpec((IN_DIM, NB), lambda n: (0, n)),
        pl.BlockSpec((N_HEADS, IN_DIM, IN_DIM), lambda n: (0, 0, 0)),
        pl.BlockSpec((8, IN_DIM), lambda n: (0, 0)),
    ]
    out_specs = [pl.BlockSpec((8, NB), lambda n: (0, n))] + [
        pl.BlockSpec((NB, 128), lambda n: (n, 0)) for _ in range(8)
    ]
    a_all, *tables = pl.pallas_call(
        _proj_body,
        grid=grid,
        in_specs=in_specs,
        out_specs=out_specs,
        out_shape=out_shapes,
    )(x_pad, shrink, v)
    return a_all, tables


def _tc_finalize(acc):
    return pl.pallas_call(
        _final_body,
        grid=(8, NPB),
        in_specs=[pl.BlockSpec((4, NB, 32), lambda f, n: (f, n, 0))],
        out_specs=pl.BlockSpec((128, NB), lambda f, n: (f, n)),
        out_shape=jax.ShapeDtypeStruct((8 * 128, NPAD), jnp.float32),
    )(acc)


_EC = 80

_sc_mesh = plsc.VectorSubcoreMesh(core_axis_name="c", subcore_axis_name="s")


def _sc_edge_body(
    rows_hbm, cols_hbm, a_hbm,
    t0, t1, t2, t3, t4, t5, t6, t7,
    acc_out,
    ridx, cidx, cidx4, exb, a0loc, a1loc, gbuf, gb4, zacc,
    den_blk,
    acc_sh, sem,
    gs0, gs1, ss0, ss1,
):
    tables = (t0, t1, t2, t3, t4, t5, t6, t7)
    gsems = (gs0, gs1)
    ssems = (ss0, ss1)
    c = lax.axis_index("c")
    s = lax.axis_index("s")

    pltpu.sync_copy(rows_hbm.at[s], ridx)
    pltpu.sync_copy(cols_hbm.at[s], cidx)

    zero16f = jnp.zeros((16,), jnp.float32)
    for i in range(16):
        for j in range(2):
            zacc[i, pl.ds(j * 16, 16)] = zero16f

    for hl in range(2):
        h = 2 * c + hl
        pltpu.sync_copy(a_hbm.at[h], a0loc)
        pltpu.sync_copy(a_hbm.at[N_HEADS + h], a1loc)
        hsplat = jnp.full((16,), hl, jnp.int32)

        def zden_body(t, carry):
            pltpu.sync_copy(zacc, acc_sh.at[pl.ds(s * 640 + t * 16, 16)])
            return carry

        lax.fori_loop(0, 40, zden_body, 0)
        plsc.subcore_barrier()

        def p1_chunk(k, carry, hl=hl, hsplat=hsplat):
            for j in range(8):
                sl = pl.ds(j * 16, 16)
                vc = cidx[k, sl]
                vr = ridx[k, sl]
                va0 = plsc.load_gather(a0loc, [vc])
                va1 = plsc.load_gather(a1loc, [vr])
                exb[hl, pl.ds(k * 128 + j * 16, 16)] = jnp.exp(va0 + va1)
            def splat_body(e, c2, hl=hl, hsplat=hsplat, k=k):
                for u in range(4):
                    w = plsc.load_gather(
                        exb,
                        [hsplat, jnp.full((16,), k * 128 + e * 4 + u, jnp.int32)],
                    )
                    gbuf[e * 4 + u, pl.ds(0, 16)] = w
                    gbuf[e * 4 + u, pl.ds(16, 16)] = w
                return c2

            lax.fori_loop(0, 32, splat_body, 0)
            pltpu.sync_copy(gbuf, acc_sh.at[ridx.at[k]], add=True)
            return carry

        lax.fori_loop(0, _EC, p1_chunk, 0)
        plsc.subcore_barrier()
        for w in range(5):
            pltpu.sync_copy(acc_sh.at[pl.ds(s * 640 + w * 128, 128)], gbuf)

            def dsave_body(r, carry, w=w, hl=hl):
                for u in range(4):
                    den_blk[hl, w * 128 + r * 4 + u, :] = gbuf[
                        r * 4 + u, pl.ds(0, 16)
                    ]
                return carry

            lax.fori_loop(0, 32, dsave_body, 0)
        plsc.subcore_barrier()

    for fb in range(8):
        cc = fb // 4
        hl = (fb % 4) // 2
        tref = tables[fb]

        @pl.when(c == cc)
        def _(fb=fb, hl=hl, tref=tref):
            def h2_body(h2, outer_carry, fb=fb, hl=hl, tref=tref):

                def prep_body(k, carry, h2=h2):
                    for j in range(8):
                        sl = pl.ds(j * 16, 16)
                        cidx4[k, sl] = cidx[k, sl] * 4 + h2
                    return carry

                lax.fori_loop(0, _EC, prep_body, 0)

                def z_body(t, carry):
                    pltpu.sync_copy(
                        zacc, acc_sh.at[pl.ds(s * 640 + t * 16, 16)]
                    )
                    return carry

                lax.fori_loop(0, 40, z_body, 0)
                plsc.subcore_barrier()
                hsplat = jnp.full((16,), hl, jnp.int32)

                for b in range(2):
                    pltpu.async_copy(
                        tref.at[cidx4.at[b]], gb4.at[b], gsems[b]
                    )

                def ring_body(g, carry, hl=hl, tref=tref, hsplat=hsplat):
                    for b in range(2):
                        k = g * 2 + b
                        pltpu.make_async_copy(
                            tref.at[cidx4.at[k]], gb4.at[b], gsems[b]
                        ).wait()

                        def e_body(e, c2, b=b, k=k):
                            for u in range(4):
                                w = plsc.load_gather(
                                    exb,
                                    [
                                        hsplat,
                                        jnp.full(
                                            (16,),
                                            k * 128 + e * 4 + u,
                                            jnp.int32,
                                        ),
                                    ],
                                )
                                for j in range(2):
                                    sl = pl.ds(j * 16, 16)
                                    gb4[b, e * 4 + u, sl] = (
                                        gb4[b, e * 4 + u, sl] * w
                                    )
                            return c2

                        lax.fori_loop(0, 32, e_body, 0)
                        pltpu.async_copy(
                            gb4.at[b], acc_sh.at[ridx.at[k]], ssems[b],
                            add=True,
                        )

                    @pl.when(g < _EC // 2 - 1)
                    def _(g=g):
                        for b in range(2):
                            k2 = (g + 1) * 2 + b
                            pltpu.make_async_copy(
                                gb4.at[b], acc_sh.at[ridx.at[k2 - 2]], ssems[b]
                            ).wait()
                            pltpu.async_copy(
                                tref.at[cidx4.at[k2]], gb4.at[b], gsems[b]
                            )

                    return carry

                lax.fori_loop(0, _EC // 2, ring_body, 0)
                for b in range(2):
                    pltpu.make_async_copy(
                        gb4.at[b], acc_sh.at[ridx.at[_EC - 2 + b]], ssems[b]
                    ).wait()
                plsc.subcore_barrier()

                for w in range(5):
                    pltpu.sync_copy(
                        acc_sh.at[pl.ds(s * 640 + w * 128, 128)], gbuf
                    )

                    def nrm_body(r, carry, w=w, hl=hl):
                        for u in range(4):
                            den16 = den_blk[hl, w * 128 + r * 4 + u, :]
                            inv = jnp.where(den16 > 0, 1.0 / den16, 0.0)
                            for j in range(2):
                                sl = pl.ds(j * 16, 16)
                                gbuf[r * 4 + u, sl] = gbuf[r * 4 + u, sl] * inv
                        return carry

                    lax.fori_loop(0, 32, nrm_body, 0)
                    pltpu.sync_copy(
                        gbuf,
                        acc_out.at[
                            4 * fb + h2, pl.ds(s * 640 + w * 128, 128)
                        ],
                    )
                plsc.subcore_barrier()
                return outer_carry

            lax.fori_loop(0, 4, h2_body, 0)


_sc_edge = functools.partial(
    pl.kernel,
    out_type=jax.ShapeDtypeStruct((32, NPAD, 32), jnp.float32),
    mesh=_sc_mesh,
    compiler_params=pltpu.CompilerParams(
        needs_layout_passes=False, use_tc_tiling_on_sc=False
    ),
    scratch_types=[
        pltpu.VMEM((_EC, 128), jnp.int32),
        pltpu.VMEM((_EC, 128), jnp.int32),
        pltpu.VMEM((_EC, 128), jnp.int32),
        pltpu.VMEM((2, NPAD), jnp.float32),
        pltpu.VMEM((NPAD,), jnp.float32),
        pltpu.VMEM((NPAD,), jnp.float32),
        pltpu.VMEM((128, 32), jnp.float32),
        pltpu.VMEM((2, 128, 32), jnp.float32),
        pltpu.VMEM((16, 32), jnp.float32),
        pltpu.VMEM((2, 640, 16), jnp.float32),
        pltpu.VMEM_SHARED((NPAD, 32), jnp.float32),
        pltpu.SemaphoreType.DMA,
        pltpu.SemaphoreType.DMA,
        pltpu.SemaphoreType.DMA,
        pltpu.SemaphoreType.DMA,
        pltpu.SemaphoreType.DMA,
    ],
)(_sc_edge_body)


def kernel(x, edge_index, shrink, att0, att1):
    x = x.astype(jnp.float32)
    x_pad = jnp.pad(x, ((0, 0), (0, NPAD - N)))
    rows = edge_index[0].astype(jnp.int32)
    cols = edge_index[1].astype(jnp.int32)
    pad_idx = N + (jnp.arange(EPAD - E, dtype=jnp.int32) % (NPAD - N))
    rows_p = jnp.concatenate([rows, pad_idx]).reshape(16, _EC, 128)
    cols_p = jnp.concatenate([cols, pad_idx]).reshape(16, _EC, 128)

    a_all, tables = _tc_project(x_pad, shrink, att0, att1)
    tables2 = [t.reshape(4 * NPAD, 32) for t in tables]
    acc = _sc_edge(rows_p, cols_p, a_all, *tables2)
    out = _tc_finalize(acc)
    return out[:, :N]


# speedup vs baseline: 8.6131x; 1.0318x over previous
import functools

import jax
import jax.numpy as jnp
from jax import lax
from jax.experimental import pallas as pl
from jax.experimental.pallas import tpu as pltpu
from jax.experimental.pallas import tpu_sc as plsc

N = 10000
NPAD = 10240
E = 160000
EPAD = 163840
IN_DIM = 256
OUT_DIM = 256
N_HEADS = 4
NB = 1024
NPB = NPAD // NB


def _v_body(att0_ref, att1_ref, shrink_ref, v_ref):
    for h in range(N_HEADS):
        v_ref[h : h + 1, :] = att0_ref[h : h + 1, :] @ shrink_ref[h]
        v_ref[N_HEADS + h : N_HEADS + h + 1, :] = att1_ref[h : h + 1, :] @ shrink_ref[h]


def _proj_body(x_ref, shrink_ref, v_ref, a_ref, *table_refs):
    xb = x_ref[...]
    a_ref[...] = v_ref[...] @ xb
    for fb in range(8):
        h, half = fb // 2, fb % 2
        w = shrink_ref[h, half * 128 : (half + 1) * 128, :]
        table_refs[fb][...] = lax.dot_general(
            xb, w, (((0,), (1,)), ((), ()))
        )


def _final_body(acc_ref, o_ref):
    a4 = acc_ref[...]
    z = jnp.concatenate([a4[0], a4[1], a4[2], a4[3]], axis=1)
    z = jnp.transpose(z, (1, 0))
    o_ref[...] = jnp.where(z > 0, z, jnp.exp(z) - 1.0)


def _tc_project(x_pad, shrink, att0, att1):
    v = pl.pallas_call(
        _v_body,
        out_shape=jax.ShapeDtypeStruct((8, IN_DIM), jnp.float32),
    )(att0, att1, shrink)

    out_shapes = (
        jax.ShapeDtypeStruct((8, NPAD), jnp.float32),
    ) + tuple(jax.ShapeDtypeStruct((NPAD, 128), jnp.float32) for _ in range(8))
    grid = (NPB,)
    in_specs = [
        pl.BlockSpec((IN_DIM, NB), lambda n: (0, n)),
        pl.BlockSpec((N_HEADS, IN_DIM, IN_DIM), lambda n: (0, 0, 0)),
        pl.BlockSpec((8, IN_DIM), lambda n: (0, 0)),
    ]
    out_specs = [pl.BlockSpec((8, NB), lambda n: (0, n))] + [
        pl.BlockSpec((NB, 128), lambda n: (n, 0)) for _ in range(8)
    ]
    a_all, *tables = pl.pallas_call(
        _proj_body,
        grid=grid,
        in_specs=in_specs,
        out_specs=out_specs,
        out_shape=out_shapes,
    )(x_pad, shrink, v)
    return a_all, tables


def _tc_finalize(acc):
    return pl.pallas_call(
        _final_body,
        grid=(8, NPB),
        in_specs=[pl.BlockSpec((4, NB, 32), lambda f, n: (f, n, 0))],
        out_specs=pl.BlockSpec((128, NB), lambda f, n: (f, n)),
        out_shape=jax.ShapeDtypeStruct((8 * 128, NPAD), jnp.float32),
    )(acc)


_EC = 80

_sc_mesh = plsc.VectorSubcoreMesh(core_axis_name="c", subcore_axis_name="s")


def _sc_edge_body(
    rows_hbm, cols_hbm, a_hbm,
    t0, t1, t2, t3, t4, t5, t6, t7,
    acc_out,
    ridx, cidx, cidx4, exb, a0loc, a1loc, gbuf, gb4, zacc,
    den_blk,
    acc_sh, sem,
    gs0, gs1, ss0, ss1,
):
    tables = (t0, t1, t2, t3, t4, t5, t6, t7)
    gsems = (gs0, gs1)
    ssems = (ss0, ss1)
    c = lax.axis_index("c")
    s = lax.axis_index("s")

    pltpu.sync_copy(rows_hbm.at[s], ridx)
    pltpu.sync_copy(cols_hbm.at[s], cidx)

    zero16f = jnp.zeros((16,), jnp.float32)

    def zfill_body(i, carry):
        zacc[i, pl.ds(0, 16)] = zero16f
        zacc[i, pl.ds(16, 16)] = zero16f
        return carry

    lax.fori_loop(0, 128, zfill_body, 0)

    for hl in range(2):
        h = 2 * c + hl
        pltpu.sync_copy(a_hbm.at[h], a0loc)
        pltpu.sync_copy(a_hbm.at[N_HEADS + h], a1loc)
        hsplat = jnp.full((16,), hl, jnp.int32)

        for t in range(5):
            pltpu.async_copy(
                zacc, acc_sh.at[pl.ds(s * 640 + t * 128, 128)], sem
            )
        for t in range(5):
            pltpu.make_async_copy(
                zacc, acc_sh.at[pl.ds(s * 640 + t * 128, 128)], sem
            ).wait()
        plsc.subcore_barrier()

        def p1_chunk(m, carry, hl=hl, hsplat=hsplat):
            for b in range(2):
                k = m * 2 + b
                for j in range(8):
                    sl = pl.ds(j * 16, 16)
                    vc = cidx[k, sl]
                    vr = ridx[k, sl]
                    va0 = plsc.load_gather(a0loc, [vc])
                    va1 = plsc.load_gather(a1loc, [vr])
                    exb[hl, pl.ds(k * 128 + j * 16, 16)] = jnp.exp(va0 + va1)

                @pl.when(m > 0)
                def _(b=b, k=k):
                    pltpu.make_async_copy(
                        gb4.at[b], acc_sh.at[ridx.at[k - 2]], ssems[b]
                    ).wait()

                def splat_body(e, c2, hl=hl, hsplat=hsplat, k=k, b=b):
                    for u in range(4):
                        w = plsc.load_gather(
                            exb,
                            [
                                hsplat,
                                jnp.full(
                                    (16,), k * 128 + e * 4 + u, jnp.int32
                                ),
                            ],
                        )
                        gb4[b, e * 4 + u, pl.ds(0, 16)] = w
                        gb4[b, e * 4 + u, pl.ds(16, 16)] = w
                    return c2

                lax.fori_loop(0, 32, splat_body, 0)
                pltpu.async_copy(
                    gb4.at[b], acc_sh.at[ridx.at[k]], ssems[b], add=True
                )
            return carry

        lax.fori_loop(0, _EC // 2, p1_chunk, 0)
        for b in range(2):
            pltpu.make_async_copy(
                gb4.at[b], acc_sh.at[ridx.at[_EC - 2 + b]], ssems[b]
            ).wait()
        plsc.subcore_barrier()
        for w in range(5):
            pltpu.sync_copy(acc_sh.at[pl.ds(s * 640 + w * 128, 128)], gbuf)

            def dsave_body(r, carry, w=w, hl=hl):
                for u in range(4):
                    den_blk[hl, w * 128 + r * 4 + u, :] = gbuf[
                        r * 4 + u, pl.ds(0, 16)
                    ]
                return carry

            lax.fori_loop(0, 32, dsave_body, 0)
        plsc.subcore_barrier()

    for fb in range(8):
        cc = fb // 4
        hl = (fb % 4) // 2
        tref = tables[fb]

        @pl.when(c == cc)
        def _(fb=fb, hl=hl, tref=tref):
            def h2_body(h2, outer_carry, fb=fb, hl=hl, tref=tref):

                def prep_body(k, carry, h2=h2):
                    for j in range(8):
                        sl = pl.ds(j * 16, 16)
                        cidx4[k, sl] = cidx[k, sl] * 4 + h2
                    return carry

                lax.fori_loop(0, _EC, prep_body, 0)

                for t in range(5):
                    pltpu.async_copy(
                        zacc, acc_sh.at[pl.ds(s * 640 + t * 128, 128)], sem
                    )
                for t in range(5):
                    pltpu.make_async_copy(
                        zacc, acc_sh.at[pl.ds(s * 640 + t * 128, 128)], sem
                    ).wait()
                plsc.subcore_barrier()
                hsplat = jnp.full((16,), hl, jnp.int32)

                for b in range(2):
                    pltpu.async_copy(
                        tref.at[cidx4.at[b]], gb4.at[b], gsems[b]
                    )

                def ring_body(g, carry, hl=hl, tref=tref, hsplat=hsplat):
                    for b in range(2):
                        k = g * 2 + b
                        pltpu.make_async_copy(
                            tref.at[cidx4.at[k]], gb4.at[b], gsems[b]
                        ).wait()

                        def e_body(e, c2, b=b, k=k):
                            for u in range(4):
                                w = plsc.load_gather(
                                    exb,
                                    [
                                        hsplat,
                                        jnp.full(
                                            (16,),
                                            k * 128 + e * 4 + u,
                                            jnp.int32,
                                        ),
                                    ],
                                )
                                for j in range(2):
                                    sl = pl.ds(j * 16, 16)
                                    gb4[b, e * 4 + u, sl] = (
                                        gb4[b, e * 4 + u, sl] * w
                                    )
                            return c2

                        lax.fori_loop(0, 32, e_body, 0)
                        pltpu.async_copy(
                            gb4.at[b], acc_sh.at[ridx.at[k]], ssems[b],
                            add=True,
                        )

                    @pl.when(g < _EC // 2 - 1)
                    def _(g=g):
                        for b in range(2):
                            k2 = (g + 1) * 2 + b
                            pltpu.make_async_copy(
                                gb4.at[b], acc_sh.at[ridx.at[k2 - 2]], ssems[b]
                            ).wait()
                            pltpu.async_copy(
                                tref.at[cidx4.at[k2]], gb4.at[b], gsems[b]
                            )

                    return carry

                lax.fori_loop(0, _EC // 2, ring_body, 0)
                for b in range(2):
                    pltpu.make_async_copy(
                        gb4.at[b], acc_sh.at[ridx.at[_EC - 2 + b]], ssems[b]
                    ).wait()
                plsc.subcore_barrier()

                for w in range(5):
                    pltpu.sync_copy(
                        acc_sh.at[pl.ds(s * 640 + w * 128, 128)], gbuf
                    )

                    def nrm_body(r, carry, w=w, hl=hl):
                        for u in range(4):
                            den16 = den_blk[hl, w * 128 + r * 4 + u, :]
                            inv = jnp.where(den16 > 0, 1.0 / den16, 0.0)
                            for j in range(2):
                                sl = pl.ds(j * 16, 16)
                                gbuf[r * 4 + u, sl] = gbuf[r * 4 + u, sl] * inv
                        return carry

                    lax.fori_loop(0, 32, nrm_body, 0)
                    pltpu.sync_copy(
                        gbuf,
                        acc_out.at[
                            4 * fb + h2, pl.ds(s * 640 + w * 128, 128)
                        ],
                    )
                plsc.subcore_barrier()
                return outer_carry

            lax.fori_loop(0, 4, h2_body, 0)


_sc_edge = functools.partial(
    pl.kernel,
    out_type=jax.ShapeDtypeStruct((32, NPAD, 32), jnp.float32),
    mesh=_sc_mesh,
    compiler_params=pltpu.CompilerParams(
        needs_layout_passes=False, use_tc_tiling_on_sc=False
    ),
    scratch_types=[
        pltpu.VMEM((_EC, 128), jnp.int32),
        pltpu.VMEM((_EC, 128), jnp.int32),
        pltpu.VMEM((_EC, 128), jnp.int32),
        pltpu.VMEM((2, NPAD), jnp.float32),
        pltpu.VMEM((NPAD,), jnp.float32),
        pltpu.VMEM((NPAD,), jnp.float32),
        pltpu.VMEM((128, 32), jnp.float32),
        pltpu.VMEM((2, 128, 32), jnp.float32),
        pltpu.VMEM((128, 32), jnp.float32),
        pltpu.VMEM((2, 640, 16), jnp.float32),
        pltpu.VMEM_SHARED((NPAD, 32), jnp.float32),
        pltpu.SemaphoreType.DMA,
        pltpu.SemaphoreType.DMA,
        pltpu.SemaphoreType.DMA,
        pltpu.SemaphoreType.DMA,
        pltpu.SemaphoreType.DMA,
    ],
)(_sc_edge_body)


def kernel(x, edge_index, shrink, att0, att1):
    x = x.astype(jnp.float32)
    x_pad = jnp.pad(x, ((0, 0), (0, NPAD - N)))
    rows = edge_index[0].astype(jnp.int32)
    cols = edge_index[1].astype(jnp.int32)
    pad_idx = N + (jnp.arange(EPAD - E, dtype=jnp.int32) % (NPAD - N))
    rows_p = jnp.concatenate([rows, pad_idx]).reshape(16, _EC, 128)
    cols_p = jnp.concatenate([cols, pad_idx]).reshape(16, _EC, 128)

    a_all, tables = _tc_project(x_pad, shrink, att0, att1)
    tables2 = [t.reshape(4 * NPAD, 32) for t in tables]
    acc = _sc_edge(rows_p, cols_p, a_all, *tables2)
    out = _tc_finalize(acc)
    return out[:, :N]
